# baseline (speedup 1.0000x reference)
import functools
import math

import jax
import jax.numpy as jnp
from jax import lax
from jax.experimental import pallas as pl
from jax.experimental.pallas import tpu as pltpu
from jax.experimental.pallas import tpu_sc as plsc

VOCAB = 1000000
D = 64
S = 200
B = 4096

NC = 2
NS = 16
NW = NC * NS

BC = B // NW
NBUF = 4
LANES = 16
SCALE = math.sqrt(D)


def _pos_encoding():
    position = jnp.arange(0, S, dtype=jnp.float32)[:, None]
    div_term = jnp.exp(
        jnp.arange(0, D, 2, dtype=jnp.float32) * -(math.log(10000.0) / D)
    )
    pe = jnp.zeros((S, D), dtype=jnp.float32)
    pe = pe.at[:, 0::2].set(jnp.sin(position * div_term))
    pe = pe.at[:, 1::2].set(jnp.cos(position * div_term))
    return pe


def _make_sc_kernel():
    mesh = plsc.VectorSubcoreMesh(core_axis_name="c", subcore_axis_name="s")

    @functools.partial(
        pl.kernel,
        out_type=jax.ShapeDtypeStruct((S, D // 8, NW, 8, BC), jnp.float32),
        mesh=mesh,
        scratch_types=[
            pltpu.VMEM((S, BC), jnp.int32),
            pltpu.VMEM((S, D), jnp.float32),
            [pltpu.VMEM((BC, D), jnp.float32)] * NBUF,
            [pltpu.VMEM((D // 8, 8, BC), jnp.float32)] * 2,
            [pltpu.SemaphoreType.DMA] * NBUF,
            [pltpu.SemaphoreType.DMA] * 2,
        ],
        compiler_params=pltpu.CompilerParams(
            use_tc_tiling_on_sc=False, needs_layout_passes=False
        ),
    )
    def sc_body(table_hbm, idx_hbm, pe_hbm, out_hbm, idx_v, pe_v, rows, tr, gsem, osem):
        wid = lax.axis_index("s") * NC + lax.axis_index("c")
        pltpu.sync_copy(idx_hbm.at[wid], idx_v)
        pltpu.sync_copy(pe_hbm, pe_v)

        iota = lax.iota(jnp.int32, LANES)

        def gather(k, b):
            return pltpu.make_async_copy(
                table_hbm.at[idx_v.at[k]], rows[b], gsem[b]
            )

        def writeout(k, tb):
            return pltpu.make_async_copy(
                tr[tb], out_hbm.at[k, slice(None), wid], osem[tb]
            )

        gather(0, 0).start()
        gather(1, 1).start()

        @pl.loop(0, S, step=NBUF)
        def _(c):
            for b in range(NBUF):
                k = c + b
                tb = b % 2

                @pl.when(k + 2 < S)
                def _():
                    gather(k + 2, (b + 2) % NBUF).start()

                gather(k, b).wait()

                @pl.when(k >= 2)
                def _():
                    writeout(k - 2, tb).wait()

                rows_b = rows[b]
                tr_b = tr[tb]
                s_vec = jnp.broadcast_to(k, (LANES,))

                @plsc.parallel_loop(0, D, unroll=2)
                def _(d):
                    d_vec = jnp.broadcast_to(d, (LANES,))
                    pe_b = plsc.load_gather(pe_v, [s_vec, d_vec])
                    dt = d >> 3
                    dr = d & 7
                    for seg in range(BC // LANES):
                        row_idx = seg * LANES + iota
                        v = plsc.load_gather(rows_b, [row_idx, d_vec])
                        tr_b[dt, dr, pl.ds(seg * LANES, LANES)] = v * SCALE + pe_b

                writeout(k, tb).start()

        writeout(S - 2, 0).wait()
        writeout(S - 1, 1).wait()

    return sc_body


_sc_kernel = _make_sc_kernel()


def kernel(input_token_ids, token_embedding):
    idx = (
        input_token_ids.astype(jnp.int32)
        .T.reshape(S, NW, BC)
        .transpose(1, 0, 2)
    )
    pe = _pos_encoding()
    out5 = _sc_kernel(token_embedding, idx, pe)
    return out5.transpose(2, 4, 0, 1, 3).reshape(B, S, D)


# speedup vs baseline: 1.5010x; 1.5010x over previous
import functools
import math

import jax
import jax.numpy as jnp
from jax import lax
from jax.experimental import pallas as pl
from jax.experimental.pallas import tpu as pltpu
from jax.experimental.pallas import tpu_sc as plsc

VOCAB = 1000000
D = 64
S = 200
B = 4096

NC = 2
NS = 16
NW = NC * NS

BC = B // NW
NBUF = 4
LANES = 16
SCALE = math.sqrt(D)


def _pos_encoding():
    position = jnp.arange(0, S, dtype=jnp.float32)[:, None]
    div_term = jnp.exp(
        jnp.arange(0, D, 2, dtype=jnp.float32) * -(math.log(10000.0) / D)
    )
    pe = jnp.zeros((S, D), dtype=jnp.float32)
    pe = pe.at[:, 0::2].set(jnp.sin(position * div_term))
    pe = pe.at[:, 1::2].set(jnp.cos(position * div_term))
    return pe


def _make_sc_kernel():
    mesh = plsc.VectorSubcoreMesh(core_axis_name="c", subcore_axis_name="s")

    @functools.partial(
        pl.kernel,
        out_type=jax.ShapeDtypeStruct((S, D // 8, NW, 8, BC), jnp.float32),
        mesh=mesh,
        scratch_types=[
            pltpu.VMEM((S, BC), jnp.int32),
            pltpu.VMEM((S, D), jnp.float32),
            [pltpu.VMEM((BC, D), jnp.float32)] * NBUF,
            [pltpu.VMEM((D // 8, 8, BC), jnp.float32)] * 2,
            [pltpu.SemaphoreType.DMA] * NBUF,
            [pltpu.SemaphoreType.DMA] * 2,
        ],
        compiler_params=pltpu.CompilerParams(
            use_tc_tiling_on_sc=False, needs_layout_passes=False
        ),
    )
    def sc_body(table_hbm, idx_hbm, pe_hbm, out_hbm, idx_v, pe_v, rows, tr, gsem, osem):
        wid = lax.axis_index("s") * NC + lax.axis_index("c")
        pltpu.sync_copy(idx_hbm.at[wid], idx_v)
        pltpu.sync_copy(pe_hbm, pe_v)

        iota = lax.iota(jnp.int32, LANES)

        def gather(k, b):
            return pltpu.make_async_copy(
                table_hbm.at[idx_v.at[k]], rows[b], gsem[b]
            )

        def writeout(k, tb):
            return pltpu.make_async_copy(
                tr[tb], out_hbm.at[k, slice(None), wid], osem[tb]
            )

        gather(0, 0).start()
        gather(1, 1).start()

        @pl.loop(0, S, step=NBUF)
        def _(c):
            for b in range(NBUF):
                k = c + b
                tb = b % 2

                @pl.when(k + 2 < S)
                def _():
                    gather(k + 2, (b + 2) % NBUF).start()

                gather(k, b).wait()

                @pl.when(k >= 2)
                def _():
                    writeout(k - 2, tb).wait()

                rows_b = rows[b]
                tr_b = tr[tb]
                s_vec = jnp.broadcast_to(k, (LANES,))

                for col0 in range(0, D, LANES):

                    @plsc.parallel_loop(0, LANES, unroll=2)
                    def _(t):
                        col_idx = ((iota + t) & (LANES - 1)) + col0
                        pe_b = plsc.load_gather(pe_v, [s_vec, col_idx])
                        dt_v = col_idx >> 3
                        dr_v = col_idx & 7
                        for r0 in range(0, BC, LANES):
                            row_idx = r0 + iota
                            v = plsc.load_gather(rows_b, [row_idx, col_idx])
                            plsc.store_scatter(
                                tr_b, [dt_v, dr_v, row_idx], v * SCALE + pe_b
                            )

                writeout(k, tb).start()

        writeout(S - 2, 0).wait()
        writeout(S - 1, 1).wait()

    return sc_body


_sc_kernel = _make_sc_kernel()


def kernel(input_token_ids, token_embedding):
    idx = (
        input_token_ids.astype(jnp.int32)
        .T.reshape(S, NW, BC)
        .transpose(1, 0, 2)
    )
    pe = _pos_encoding()
    out5 = _sc_kernel(token_embedding, idx, pe)
    return out5.transpose(2, 4, 0, 1, 3).reshape(B, S, D)
